# baseline (speedup 1.0000x reference)
import jax
import jax.numpy as jnp
from jax import lax
from jax.experimental import pallas as pl
from jax.experimental.pallas import tpu as pltpu
from jax.experimental.pallas import tpu_sc as plsc

BS, K, D = 16384, 50, 64
L = 16
NC, NS = 2, 16
NW = NC * NS
NBT = BS // 128
BT_PER_W = NBT // NW
NG = BT_PER_W * 8
PEEL = 11
NEG = float("-inf")


def _body(i_hbm, c_hbm, n_hbm, out_hbm,
          ibuf0, ibuf1, cbuf0, cbuf1, nbuf0, nbuf1, wbuf, mwbuf,
          obuf0, obuf1, isem0, isem1, osem0, osem1):
    wid = lax.axis_index("s") * NC + lax.axis_index("c")
    negv = jnp.full((L,), NEG, dtype=jnp.float32)
    posv = jnp.full((L,), float("inf"), dtype=jnp.float32)
    zi = jnp.zeros((L,), jnp.int32)
    zf = jnp.zeros((L,), jnp.float32)

    ibufs, cbufs, nbufs = (ibuf0, ibuf1), (cbuf0, cbuf1), (nbuf0, nbuf1)
    obufs, isems, osems = (obuf0, obuf1), (isem0, isem1), (osem0, osem1)

    def addr(g):
        bt = wid * BT_PER_W + (g >> 3)
        bs0 = (g & 7) * L
        return bt, bs0

    def start_in(g, s):
        bt, bs0 = addr(g)
        pltpu.async_copy(i_hbm.at[:, :, bt, :, pl.ds(bs0, L)], ibufs[s],
                         isems[s])
        pltpu.async_copy(c_hbm.at[:, bt, :, pl.ds(bs0, L)], cbufs[s],
                         isems[s])
        pltpu.async_copy(n_hbm.at[pl.ds(bt * 128 + bs0, L)], nbufs[s],
                         isems[s])

    def wait_in(g, s):
        bt, bs0 = addr(g)
        pltpu.make_async_copy(i_hbm.at[:, :, bt, :, pl.ds(bs0, L)], ibufs[s],
                              isems[s]).wait()
        pltpu.make_async_copy(c_hbm.at[:, bt, :, pl.ds(bs0, L)], cbufs[s],
                              isems[s]).wait()
        pltpu.make_async_copy(n_hbm.at[pl.ds(bt * 128 + bs0, L)], nbufs[s],
                              isems[s]).wait()

    def wait_out(g, s):
        bt, bs0 = addr(g)
        pltpu.make_async_copy(obufs[s], out_hbm.at[:, bt, :, pl.ds(bs0, L)],
                              osems[s]).wait()

    def compute(g, s):
        ibuf, cbuf, cnt_ref = ibufs[s], cbufs[s], nbufs[s]
        obuf = obufs[s]

        for dc in range(4):
            cd = [cbuf[(dc * L + j) // 8, (dc * L + j) % 8, :]
                  for j in range(L)]

            @plsc.parallel_loop(0, K, 1, unroll=4)
            def kA(k, dc=dc, cd=cd):
                p = [ibuf[k, (dc * L + j) // 8, (dc * L + j) % 8, :] * cd[j]
                     for j in range(L)]
                while len(p) > 1:
                    p = [p[i] + p[i + 1] for i in range(0, len(p) - 1, 2)] \
                        + ([p[-1]] if len(p) & 1 else [])
                if dc == 0:
                    wbuf[k, :] = p[0]
                else:
                    wbuf[k, :] = wbuf[k, :] + p[0]

        cnt = cnt_ref[:]
        x = (cnt * 10).astype(jnp.float32)
        bits = lax.bitcast_convert_type(x, jnp.int32)
        e = ((bits >> 23) & 0xFF) - 127
        frac = bits & 0x7FFFFF
        dk = jnp.clip(e + (frac != 0).astype(jnp.int32), 1, K)

        def ins(k, s):
            v = wbuf[k, :]
            out = []
            for i in range(PEEL):
                out.append(jnp.maximum(s[i], v))
                v = jnp.minimum(s[i], v)
            return tuple(out)

        slist = lax.fori_loop(0, K, ins, (negv,) * PEEL, unroll=2)
        Th = negv
        for i in range(PEEL):
            Th = jnp.where(dk == i + 1, slist[i], Th)

        def gk(k, a):
            return a + (wbuf[k, :] > Th).astype(jnp.int32)

        rext = dk - lax.fori_loop(0, K, gk, zi, unroll=5)

        def mwk(k, eq):
            v = wbuf[k, :]
            iseq = (v == Th)
            keep = (v > Th) | (iseq & (eq < rext))
            mwbuf[k, :] = jnp.where(keep, v, jnp.float32(0.0))
            return eq + iseq.astype(jnp.int32)

        lax.fori_loop(0, K, mwk, zi, unroll=5)

        for dc in range(4):
            def kD(k, acc, dc=dc):
                mwk_ = mwbuf[k, :]
                return tuple(
                    acc[j] + ibuf[k, (dc * L + j) // 8, (dc * L + j) % 8, :]
                    * mwk_
                    for j in range(L))

            acc = plsc.parallel_loop(0, K, 1, unroll=4, carry=(zf,) * L)(kD)
            for j in range(L):
                obuf[(dc * L + j) // 8, (dc * L + j) % 8, :] = acc[j]

        bt, bs0 = addr(g)
        pltpu.async_copy(obuf, out_hbm.at[:, bt, :, pl.ds(bs0, L)], osems[s])

    start_in(0, 0)
    start_in(1, 1)

    def pair(p, _):
        for s in (0, 1):
            g = p * 2 + s
            wait_in(g, s)

            @pl.when(g >= 2)
            def _():
                wait_out(g - 2, s)

            compute(g, s)

            @pl.when(g + 2 < NG)
            def _():
                start_in(g + 2, s)
        return 0

    lax.fori_loop(0, NG // 2, pair, 0)
    wait_out(NG - 2, 0)
    wait_out(NG - 1, 1)


_sc_call = pl.kernel(
    _body,
    out_type=jax.ShapeDtypeStruct((8, 128, 8, 128), jnp.float32),
    mesh=plsc.VectorSubcoreMesh(core_axis_name="c", subcore_axis_name="s",
                                num_cores=NC, num_subcores=NS),
    compiler_params=pltpu.CompilerParams(needs_layout_passes=False,
                                         use_tc_tiling_on_sc=False),
    scratch_types=[
        pltpu.VMEM((K, 8, 8, L), jnp.float32),
        pltpu.VMEM((K, 8, 8, L), jnp.float32),
        pltpu.VMEM((8, 8, L), jnp.float32),
        pltpu.VMEM((8, 8, L), jnp.float32),
        pltpu.VMEM((L,), jnp.int32),
        pltpu.VMEM((L,), jnp.int32),
        pltpu.VMEM((K, L), jnp.float32),
        pltpu.VMEM((K, L), jnp.float32),
        pltpu.VMEM((8, 8, L), jnp.float32),
        pltpu.VMEM((8, 8, L), jnp.float32),
        pltpu.SemaphoreType.DMA,
        pltpu.SemaphoreType.DMA,
        pltpu.SemaphoreType.DMA,
        pltpu.SemaphoreType.DMA,
    ],
)


@jax.jit
def kernel(interest_representations, candidate_news_representation,
           unique_category_counts):
    i5 = interest_representations.reshape(128, 128, K, 8, 8).transpose(
        2, 3, 0, 4, 1)
    c4 = candidate_news_representation.reshape(128, 128, 8, 8).transpose(
        2, 0, 3, 1)
    o4 = _sc_call(i5, c4, unique_category_counts)
    return o4.transpose(1, 3, 0, 2).reshape(BS, D)


# speedup vs baseline: 1.0735x; 1.0735x over previous
import jax
import jax.numpy as jnp
from jax import lax
from jax.experimental import pallas as pl
from jax.experimental.pallas import tpu as pltpu
from jax.experimental.pallas import tpu_sc as plsc

BS, K, D = 16384, 50, 64
L = 16
NC, NS = 2, 16
NW = NC * NS
NBT = BS // 128
BT_PER_W = NBT // NW
NG = BT_PER_W * 8
PEEL = 11
NEG = float("-inf")


def _body(i_hbm, c_hbm, n_hbm, out_hbm,
          ibuf0, ibuf1, cbuf0, cbuf1, nbuf0, nbuf1, wbuf, mwbuf,
          obuf0, obuf1, isem0, isem1, osem0, osem1):
    wid = lax.axis_index("s") * NC + lax.axis_index("c")
    negv = jnp.full((L,), NEG, dtype=jnp.float32)
    posv = jnp.full((L,), float("inf"), dtype=jnp.float32)
    zi = jnp.zeros((L,), jnp.int32)
    zf = jnp.zeros((L,), jnp.float32)

    ibufs, cbufs, nbufs = (ibuf0, ibuf1), (cbuf0, cbuf1), (nbuf0, nbuf1)
    obufs, isems, osems = (obuf0, obuf1), (isem0, isem1), (osem0, osem1)

    def addr(g):
        bt = wid * BT_PER_W + (g >> 3)
        bs0 = (g & 7) * L
        return bt, bs0

    def start_in(g, s):
        bt, bs0 = addr(g)
        pltpu.async_copy(i_hbm.at[:, :, bt, :, pl.ds(bs0, L)], ibufs[s],
                         isems[s])
        pltpu.async_copy(c_hbm.at[:, bt, :, pl.ds(bs0, L)], cbufs[s],
                         isems[s])
        pltpu.async_copy(n_hbm.at[pl.ds(bt * 128 + bs0, L)], nbufs[s],
                         isems[s])

    def wait_in(g, s):
        bt, bs0 = addr(g)
        pltpu.make_async_copy(i_hbm.at[:, :, bt, :, pl.ds(bs0, L)], ibufs[s],
                              isems[s]).wait()
        pltpu.make_async_copy(c_hbm.at[:, bt, :, pl.ds(bs0, L)], cbufs[s],
                              isems[s]).wait()
        pltpu.make_async_copy(n_hbm.at[pl.ds(bt * 128 + bs0, L)], nbufs[s],
                              isems[s]).wait()

    def wait_out(g, s):
        bt, bs0 = addr(g)
        pltpu.make_async_copy(obufs[s], out_hbm.at[:, bt, :, pl.ds(bs0, L)],
                              osems[s]).wait()

    def compute(g, s):
        ibuf, cbuf, cnt_ref = ibufs[s], cbufs[s], nbufs[s]
        obuf = obufs[s]

        for dc in range(4):
            cd = [cbuf[(dc * L + j) // 8, (dc * L + j) % 8, :]
                  for j in range(L)]

            @plsc.parallel_loop(0, K, 1, unroll=2)
            def kA(k, dc=dc, cd=cd):
                p = [ibuf[k, (dc * L + j) // 8, (dc * L + j) % 8, :] * cd[j]
                     for j in range(L)]
                while len(p) > 1:
                    p = [p[i] + p[i + 1] for i in range(0, len(p) - 1, 2)] \
                        + ([p[-1]] if len(p) & 1 else [])
                if dc == 0:
                    wbuf[k, :] = p[0]
                else:
                    wbuf[k, :] = wbuf[k, :] + p[0]

        cnt = cnt_ref[:]
        x = (cnt * 10).astype(jnp.float32)
        bits = lax.bitcast_convert_type(x, jnp.int32)
        e = ((bits >> 23) & 0xFF) - 127
        frac = bits & 0x7FFFFF
        dk = jnp.clip(e + (frac != 0).astype(jnp.int32), 1, K)

        def ins(k, s):
            v = wbuf[k, :]
            out = []
            for i in range(PEEL):
                out.append(jnp.maximum(s[i], v))
                v = jnp.minimum(s[i], v)
            return tuple(out)

        slist = plsc.parallel_loop(0, K, 1, unroll=2,
                                   carry=(negv,) * PEEL)(ins)
        Th = negv
        for i in range(PEEL):
            Th = jnp.where(dk == i + 1, slist[i], Th)

        def gk(k, a):
            return a + (wbuf[k, :] > Th).astype(jnp.int32)

        rext = dk - plsc.parallel_loop(0, K, 1, unroll=5, carry=zi)(gk)

        def mwk(k, eq):
            v = wbuf[k, :]
            iseq = (v == Th)
            keep = (v > Th) | (iseq & (eq < rext))
            mwbuf[k, :] = jnp.where(keep, v, jnp.float32(0.0))
            return eq + iseq.astype(jnp.int32)

        plsc.parallel_loop(0, K, 1, unroll=5, carry=zi)(mwk)

        for dc in range(4):
            def kD(k, acc, dc=dc):
                mwk_ = mwbuf[k, :]
                return tuple(
                    acc[j] + ibuf[k, (dc * L + j) // 8, (dc * L + j) % 8, :]
                    * mwk_
                    for j in range(L))

            acc = plsc.parallel_loop(0, K, 1, unroll=2, carry=(zf,) * L)(kD)
            for j in range(L):
                obuf[(dc * L + j) // 8, (dc * L + j) % 8, :] = acc[j]

        bt, bs0 = addr(g)
        pltpu.async_copy(obuf, out_hbm.at[:, bt, :, pl.ds(bs0, L)], osems[s])

    start_in(0, 0)
    start_in(1, 1)

    def pair(p, _):
        for s in (0, 1):
            g = p * 2 + s
            wait_in(g, s)

            @pl.when(g >= 2)
            def _():
                wait_out(g - 2, s)

            compute(g, s)

            @pl.when(g + 2 < NG)
            def _():
                start_in(g + 2, s)
        return 0

    lax.fori_loop(0, NG // 2, pair, 0)
    wait_out(NG - 2, 0)
    wait_out(NG - 1, 1)


_sc_call = pl.kernel(
    _body,
    out_type=jax.ShapeDtypeStruct((8, 128, 8, 128), jnp.float32),
    mesh=plsc.VectorSubcoreMesh(core_axis_name="c", subcore_axis_name="s",
                                num_cores=NC, num_subcores=NS),
    compiler_params=pltpu.CompilerParams(needs_layout_passes=False,
                                         use_tc_tiling_on_sc=False),
    scratch_types=[
        pltpu.VMEM((K, 8, 8, L), jnp.float32),
        pltpu.VMEM((K, 8, 8, L), jnp.float32),
        pltpu.VMEM((8, 8, L), jnp.float32),
        pltpu.VMEM((8, 8, L), jnp.float32),
        pltpu.VMEM((L,), jnp.int32),
        pltpu.VMEM((L,), jnp.int32),
        pltpu.VMEM((K, L), jnp.float32),
        pltpu.VMEM((K, L), jnp.float32),
        pltpu.VMEM((8, 8, L), jnp.float32),
        pltpu.VMEM((8, 8, L), jnp.float32),
        pltpu.SemaphoreType.DMA,
        pltpu.SemaphoreType.DMA,
        pltpu.SemaphoreType.DMA,
        pltpu.SemaphoreType.DMA,
    ],
)


@jax.jit
def kernel(interest_representations, candidate_news_representation,
           unique_category_counts):
    i5 = interest_representations.reshape(128, 128, K, 8, 8).transpose(
        2, 3, 0, 4, 1)
    c4 = candidate_news_representation.reshape(128, 128, 8, 8).transpose(
        2, 0, 3, 1)
    o4 = _sc_call(i5, c4, unique_category_counts)
    return o4.transpose(1, 3, 0, 2).reshape(BS, D)


# speedup vs baseline: 1.0739x; 1.0003x over previous
import jax
import jax.numpy as jnp
from jax import lax
from jax.experimental import pallas as pl
from jax.experimental.pallas import tpu as pltpu
from jax.experimental.pallas import tpu_sc as plsc

BS, K, D = 16384, 50, 64
L = 16
NC, NS = 2, 16
NW = NC * NS
NBT = BS // 128
BT_PER_W = NBT // NW
NG = BT_PER_W * 8
PEEL = 11
NEG = float("-inf")


def _body(i_hbm, c_hbm, n_hbm, out_hbm,
          ibuf0, ibuf1, cbuf0, cbuf1, nbuf0, nbuf1, wbuf, mwbuf,
          obuf0, obuf1, isem0, isem1, osem0, osem1):
    wid = lax.axis_index("s") * NC + lax.axis_index("c")
    negv = jnp.full((L,), NEG, dtype=jnp.float32)
    zi = jnp.zeros((L,), jnp.int32)
    zf = jnp.zeros((L,), jnp.float32)

    ibufs, cbufs, nbufs = (ibuf0, ibuf1), (cbuf0, cbuf1), (nbuf0, nbuf1)
    obufs, isems, osems = (obuf0, obuf1), (isem0, isem1), (osem0, osem1)

    def addr(g):
        bt = wid * BT_PER_W + (g >> 3)
        bs0 = (g & 7) * L
        return bt, bs0

    def start_in(g, s):
        bt, bs0 = addr(g)
        pltpu.async_copy(i_hbm.at[:, :, bt, :, pl.ds(bs0, L)], ibufs[s],
                         isems[s])
        pltpu.async_copy(c_hbm.at[:, bt, :, pl.ds(bs0, L)], cbufs[s],
                         isems[s])
        pltpu.async_copy(n_hbm.at[pl.ds(bt * 128 + bs0, L)], nbufs[s],
                         isems[s])

    def wait_in(g, s):
        bt, bs0 = addr(g)
        pltpu.make_async_copy(i_hbm.at[:, :, bt, :, pl.ds(bs0, L)], ibufs[s],
                              isems[s]).wait()
        pltpu.make_async_copy(c_hbm.at[:, bt, :, pl.ds(bs0, L)], cbufs[s],
                              isems[s]).wait()
        pltpu.make_async_copy(n_hbm.at[pl.ds(bt * 128 + bs0, L)], nbufs[s],
                              isems[s]).wait()

    def wait_out(g, s):
        bt, bs0 = addr(g)
        pltpu.make_async_copy(obufs[s], out_hbm.at[:, bt, :, pl.ds(bs0, L)],
                              osems[s]).wait()

    def compute(g, s):
        ibuf, cbuf, cnt_ref = ibufs[s], cbufs[s], nbufs[s]
        obuf = obufs[s]

        for dc in range(4):
            cd = [cbuf[(dc * L + j) // 8, (dc * L + j) % 8, :]
                  for j in range(L)]

            @plsc.parallel_loop(0, K, 1, unroll=2)
            def kA(k, dc=dc, cd=cd):
                p = [ibuf[k, (dc * L + j) // 8, (dc * L + j) % 8, :] * cd[j]
                     for j in range(L)]
                while len(p) > 1:
                    p = [p[i] + p[i + 1] for i in range(0, len(p) - 1, 2)] \
                        + ([p[-1]] if len(p) & 1 else [])
                if dc == 0:
                    wbuf[k, :] = p[0]
                else:
                    wbuf[k, :] = wbuf[k, :] + p[0]

        cnt = cnt_ref[:]
        x = (cnt * 10).astype(jnp.float32)
        bits = lax.bitcast_convert_type(x, jnp.int32)
        e = ((bits >> 23) & 0xFF) - 127
        frac = bits & 0x7FFFFF
        dk = jnp.clip(e + (frac != 0).astype(jnp.int32), 1, K)

        def ins(k, s):
            v = wbuf[k, :]
            out = []
            for i in range(PEEL):
                out.append(jnp.maximum(s[i], v))
                v = jnp.minimum(s[i], v)
            return tuple(out)

        slist = plsc.parallel_loop(0, K, 1, unroll=2,
                                   carry=(negv,) * PEEL)(ins)
        Th = negv
        for i in range(PEEL):
            Th = jnp.where(dk == i + 1, slist[i], Th)

        def gk(k, a):
            return a + (wbuf[k, :] > Th).astype(jnp.int32)

        rext = dk - plsc.parallel_loop(0, K, 1, unroll=5, carry=zi)(gk)

        def mwk(k, eq):
            v = wbuf[k, :]
            iseq = (v == Th)
            keep = (v > Th) | (iseq & (eq < rext))
            mwbuf[k, :] = jnp.where(keep, v, jnp.float32(0.0))
            return eq + iseq.astype(jnp.int32)

        plsc.parallel_loop(0, K, 1, unroll=5, carry=zi)(mwk)

        for dc in range(4):
            def kD(k, acc, dc=dc):
                mwk_ = mwbuf[k, :]
                return tuple(
                    acc[j] + ibuf[k, (dc * L + j) // 8, (dc * L + j) % 8, :]
                    * mwk_
                    for j in range(L))

            acc = plsc.parallel_loop(0, K, 1, unroll=2, carry=(zf,) * L)(kD)
            for j in range(L):
                obuf[(dc * L + j) // 8, (dc * L + j) % 8, :] = acc[j]

        bt, bs0 = addr(g)
        pltpu.async_copy(obuf, out_hbm.at[:, bt, :, pl.ds(bs0, L)], osems[s])

    start_in(0, 0)
    start_in(1, 1)

    def pair(p, _):
        for s in (0, 1):
            g = p * 2 + s
            wait_in(g, s)

            @pl.when(g >= 2)
            def _():
                wait_out(g - 2, s)

            compute(g, s)

            @pl.when(g + 2 < NG)
            def _():
                start_in(g + 2, s)
        return 0

    lax.fori_loop(0, NG // 2, pair, 0)
    wait_out(NG - 2, 0)
    wait_out(NG - 1, 1)


_sc_call = pl.kernel(
    _body,
    out_type=jax.ShapeDtypeStruct((8, 128, 8, 128), jnp.float32),
    mesh=plsc.VectorSubcoreMesh(core_axis_name="c", subcore_axis_name="s",
                                num_cores=NC, num_subcores=NS),
    compiler_params=pltpu.CompilerParams(needs_layout_passes=False,
                                         use_tc_tiling_on_sc=False),
    scratch_types=[
        pltpu.VMEM((K, 8, 8, L), jnp.float32),
        pltpu.VMEM((K, 8, 8, L), jnp.float32),
        pltpu.VMEM((8, 8, L), jnp.float32),
        pltpu.VMEM((8, 8, L), jnp.float32),
        pltpu.VMEM((L,), jnp.int32),
        pltpu.VMEM((L,), jnp.int32),
        pltpu.VMEM((K, L), jnp.float32),
        pltpu.VMEM((K, L), jnp.float32),
        pltpu.VMEM((8, 8, L), jnp.float32),
        pltpu.VMEM((8, 8, L), jnp.float32),
        pltpu.SemaphoreType.DMA,
        pltpu.SemaphoreType.DMA,
        pltpu.SemaphoreType.DMA,
        pltpu.SemaphoreType.DMA,
    ],
)


@jax.jit
def kernel(interest_representations, candidate_news_representation,
           unique_category_counts):
    i5 = interest_representations.reshape(128, 128, K, 8, 8).transpose(
        2, 3, 0, 4, 1)
    c4 = candidate_news_representation.reshape(128, 128, 8, 8).transpose(
        2, 0, 3, 1)
    o4 = _sc_call(i5, c4, unique_category_counts)
    return o4.transpose(1, 3, 0, 2).reshape(BS, D)
